# baseline (speedup 1.0000x reference)
import functools

import jax
import jax.numpy as jnp
from jax import lax
from jax.experimental import pallas as pl
from jax.experimental.pallas import tpu as pltpu
from jax.experimental.pallas import tpu_sc as plsc

_NUM_WORKERS = 32
_CHUNK = 128


@functools.partial(jax.jit, static_argnums=(2, 3, 4))
def _emb(idx, weight, n_total, d, n_per_w):
    n_chunks = n_per_w // _CHUNK
    mesh = plsc.VectorSubcoreMesh(core_axis_name="c", subcore_axis_name="s")

    @functools.partial(
        pl.kernel,
        out_type=jax.ShapeDtypeStruct((n_total, d), jnp.float32),
        mesh=mesh,
        scratch_types=[
            pltpu.VMEM((_CHUNK,), jnp.int32),
            pltpu.VMEM((_CHUNK, d), jnp.float32),
            pltpu.SemaphoreType.DMA,
        ],
        compiler_params=pltpu.CompilerParams(use_tc_tiling_on_sc=False),
    )
    def emb(idx_hbm, table_hbm, out_hbm, idx_v, rows_v, sem):
        wid = lax.axis_index("s") * 2 + lax.axis_index("c")
        base = wid * n_per_w

        def body(i, carry):
            off = base + i * _CHUNK
            pltpu.sync_copy(idx_hbm.at[pl.ds(off, _CHUNK)], idx_v)
            pltpu.async_copy(table_hbm.at[idx_v], rows_v, sem).wait()
            pltpu.sync_copy(rows_v, out_hbm.at[pl.ds(off, _CHUNK)])
            return carry

        lax.fori_loop(0, n_chunks, body, 0)

    return emb(idx, weight)


def kernel(x, weight):
    b, s = x.shape
    v, d = weight.shape
    n = b * s
    idx = x.reshape(n).astype(jnp.int32)
    out = _emb(idx, weight, n, d, n // _NUM_WORKERS)
    return out.reshape(b, s, d)


# speedup vs baseline: 1.1457x; 1.1457x over previous
import functools

import jax
import jax.numpy as jnp
from jax import lax
from jax.experimental import pallas as pl
from jax.experimental.pallas import tpu as pltpu
from jax.experimental.pallas import tpu_sc as plsc

_NUM_WORKERS = 32
_CHUNK = 512


@functools.partial(jax.jit, static_argnums=(2, 3, 4))
def _emb(idx, weight, n_total, d, n_per_w):
    n_chunks = n_per_w // _CHUNK
    mesh = plsc.VectorSubcoreMesh(core_axis_name="c", subcore_axis_name="s")

    @functools.partial(
        pl.kernel,
        out_type=jax.ShapeDtypeStruct((n_total, d), jnp.float32),
        mesh=mesh,
        scratch_types=[
            pltpu.VMEM((_CHUNK,), jnp.int32),
            pltpu.VMEM((_CHUNK, d), jnp.float32),
            pltpu.SemaphoreType.DMA,
        ],
        compiler_params=pltpu.CompilerParams(use_tc_tiling_on_sc=False),
    )
    def emb(idx_hbm, table_hbm, out_hbm, idx_v, rows_v, sem):
        wid = lax.axis_index("s") * 2 + lax.axis_index("c")
        base = wid * n_per_w

        def body(i, carry):
            off = base + i * _CHUNK
            pltpu.sync_copy(idx_hbm.at[pl.ds(off, _CHUNK)], idx_v)
            pltpu.async_copy(table_hbm.at[idx_v], rows_v, sem).wait()
            pltpu.sync_copy(rows_v, out_hbm.at[pl.ds(off, _CHUNK)])
            return carry

        lax.fori_loop(0, n_chunks, body, 0)

    return emb(idx, weight)


def kernel(x, weight):
    b, s = x.shape
    v, d = weight.shape
    n = b * s
    idx = x.reshape(n).astype(jnp.int32)
    out = _emb(idx, weight, n, d, n // _NUM_WORKERS)
    return out.reshape(b, s, d)


# speedup vs baseline: 1.1961x; 1.0440x over previous
import functools

import jax
import jax.numpy as jnp
from jax import lax
from jax.experimental import pallas as pl
from jax.experimental.pallas import tpu as pltpu
from jax.experimental.pallas import tpu_sc as plsc

_NUM_WORKERS = 32
_CHUNK = 256
_NBUF = 4


@functools.partial(jax.jit, static_argnums=(2, 3, 4))
def _emb(idx, weight, n_total, d, n_per_w):
    n_chunks = n_per_w // _CHUNK
    n_outer = n_chunks // _NBUF
    mesh = plsc.VectorSubcoreMesh(core_axis_name="c", subcore_axis_name="s")

    @functools.partial(
        pl.kernel,
        out_type=jax.ShapeDtypeStruct((n_total, d), jnp.float32),
        mesh=mesh,
        scratch_types=[
            pltpu.VMEM((n_per_w,), jnp.int32),
            [pltpu.VMEM((_CHUNK, d), jnp.float32) for _ in range(_NBUF)],
            [pltpu.SemaphoreType.DMA for _ in range(_NBUF)],
            [pltpu.SemaphoreType.DMA for _ in range(_NBUF)],
        ],
        compiler_params=pltpu.CompilerParams(use_tc_tiling_on_sc=False),
    )
    def emb(idx_hbm, table_hbm, out_hbm, idx_all, rows, sg, so):
        wid = lax.axis_index("s") * 2 + lax.axis_index("c")
        base = wid * n_per_w

        pltpu.sync_copy(idx_hbm.at[pl.ds(base, n_per_w)], idx_all)

        def fire_gather(b, c):
            pltpu.async_copy(
                table_hbm.at[idx_all.at[pl.ds(c * _CHUNK, _CHUNK)]],
                rows[b], sg[b])

        def wait_gather(b, c):
            pltpu.make_async_copy(
                table_hbm.at[idx_all.at[pl.ds(c * _CHUNK, _CHUNK)]],
                rows[b], sg[b]).wait()

        def fire_out(b, c):
            pltpu.async_copy(
                rows[b], out_hbm.at[pl.ds(base + c * _CHUNK, _CHUNK)], so[b])

        def wait_out(b, c):
            pltpu.make_async_copy(
                rows[b], out_hbm.at[pl.ds(base + c * _CHUNK, _CHUNK)],
                so[b]).wait()

        for b in range(_NBUF):
            fire_gather(b, b)

        def outer(g, carry):
            c0 = g * _NBUF
            for b in range(_NBUF):
                wait_gather(b, c0 + b)
                fire_out(b, c0 + b)
            for b in range(_NBUF):
                c_next = c0 + b + _NBUF

                @pl.when(c_next < n_chunks)
                def _():
                    wait_out(b, c0 + b)
                    fire_gather(b, c_next)

            return carry

        lax.fori_loop(0, n_outer, outer, 0)

        for b in range(_NBUF):
            wait_out(b, n_chunks - _NBUF + b)

    return emb(idx, weight)


def kernel(x, weight):
    b, s = x.shape
    v, d = weight.shape
    n = b * s
    idx = x.reshape(n).astype(jnp.int32)
    out = _emb(idx, weight, n, d, n // _NUM_WORKERS)
    return out.reshape(b, s, d)


# speedup vs baseline: 1.4404x; 1.2043x over previous
import functools

import jax
import jax.numpy as jnp
from jax import lax
from jax.experimental import pallas as pl
from jax.experimental.pallas import tpu as pltpu
from jax.experimental.pallas import tpu_sc as plsc

_NUM_WORKERS = 32
_CHUNK = 128
_NBUF = 4


@functools.partial(jax.jit, static_argnums=(2, 3, 4))
def _emb(idx, table, n_total, d, n_per_w):
    n_chunks = n_per_w // _CHUNK
    n_outer = n_chunks // _NBUF
    mesh = plsc.VectorSubcoreMesh(core_axis_name="c", subcore_axis_name="s")

    @functools.partial(
        pl.kernel,
        out_type=jax.ShapeDtypeStruct((n_total, 2 * d), jnp.float32),
        mesh=mesh,
        scratch_types=[
            pltpu.VMEM((n_per_w,), jnp.int32),
            [pltpu.VMEM((_CHUNK, 2 * d), jnp.float32) for _ in range(_NBUF)],
            [pltpu.SemaphoreType.DMA for _ in range(_NBUF)],
            [pltpu.SemaphoreType.DMA for _ in range(_NBUF)],
        ],
        compiler_params=pltpu.CompilerParams(use_tc_tiling_on_sc=False),
    )
    def emb(idx_hbm, table_hbm, out_hbm, idx_all, rows, sg, so):
        wid = lax.axis_index("s") * 2 + lax.axis_index("c")
        base = wid * n_per_w

        pltpu.sync_copy(idx_hbm.at[pl.ds(base, n_per_w)], idx_all)

        def fire_gather(b, c):
            pltpu.async_copy(
                table_hbm.at[idx_all.at[pl.ds(c * _CHUNK, _CHUNK)]],
                rows[b], sg[b])

        def wait_gather(b, c):
            pltpu.make_async_copy(
                table_hbm.at[idx_all.at[pl.ds(c * _CHUNK, _CHUNK)]],
                rows[b], sg[b]).wait()

        def fire_out(b, c):
            pltpu.async_copy(
                rows[b].at[:, pl.ds(0, d)],
                out_hbm.at[pl.ds(base + c * _CHUNK, _CHUNK), pl.ds(0, d)],
                so[b])

        def wait_out(b, c):
            pltpu.make_async_copy(
                rows[b].at[:, pl.ds(0, d)],
                out_hbm.at[pl.ds(base + c * _CHUNK, _CHUNK), pl.ds(0, d)],
                so[b]).wait()

        for b in range(_NBUF):
            fire_gather(b, b)

        def outer(g, carry):
            c0 = g * _NBUF
            for b in range(_NBUF):
                wait_gather(b, c0 + b)
                fire_out(b, c0 + b)
            for b in range(_NBUF):
                c_next = c0 + b + _NBUF

                @pl.when(c_next < n_chunks)
                def _():
                    wait_out(b, c0 + b)
                    fire_gather(b, c_next)

            return carry

        lax.fori_loop(0, n_outer, outer, 0)

        for b in range(_NBUF):
            wait_out(b, n_chunks - _NBUF + b)

    return emb(idx, table)


def kernel(x, weight):
    b, s = x.shape
    v, d = weight.shape
    n = b * s
    idx = x.reshape(n).astype(jnp.int32)
    table = jnp.pad(weight, ((0, 0), (0, d)))
    out = _emb(idx, table, n, d, n // _NUM_WORKERS)
    return out[:, :d].reshape(b, s, d)
